# Initial kernel scaffold; baseline (speedup 1.0000x reference)
import jax
import jax.numpy as jnp
from jax.experimental import pallas as pl


def kernel(x, edge_index, W, b):
    raise NotImplementedError("write your pallas kernel here")


# speedup vs baseline: 102.6526x; 102.6526x over previous
import functools

import jax
import jax.numpy as jnp
from jax import lax
from jax.experimental import pallas as pl
from jax.experimental.pallas import tpu as pltpu
from jax.experimental.pallas import tpu_sc as plsc

N = 10000
E = 320000
D = 128

_NC, _NS, _L = 2, 16, 16
_NW = _NC * _NS
_E_PER_W = E // _NW


def _scores_body(x_ref, w_ref, b_ref, out_ref):
    s = jnp.sum(x_ref[...] * w_ref[...], axis=1) + b_ref[0]
    out_ref[...] = jax.nn.sigmoid(s)


def _compute_scores(x, W, b):
    return pl.pallas_call(
        _scores_body,
        out_shape=jax.ShapeDtypeStruct((N,), jnp.float32),
        in_specs=[
            pl.BlockSpec(memory_space=pltpu.VMEM),
            pl.BlockSpec(memory_space=pltpu.VMEM),
            pl.BlockSpec(memory_space=pltpu.SMEM),
        ],
        out_specs=pl.BlockSpec(memory_space=pltpu.VMEM),
    )(x, W, b)


@functools.cache
def _build_edge_kernel():
    mesh = plsc.VectorSubcoreMesh(core_axis_name="c", subcore_axis_name="s")
    return pl.kernel(
        _edge_body,
        out_type=jax.ShapeDtypeStruct((E,), jnp.float32),
        mesh=mesh,
        scratch_types=[
            pltpu.VMEM((N,), jnp.float32),
            pltpu.VMEM((_E_PER_W,), jnp.int32),
            pltpu.VMEM((_E_PER_W,), jnp.int32),
            pltpu.VMEM((_E_PER_W,), jnp.float32),
            pltpu.SemaphoreType.DMA,
        ],
        compiler_params=pltpu.CompilerParams(needs_layout_passes=False),
    )


def _edge_body(scores_hbm, edge_hbm, out_hbm, scores_v, row_v, col_v, out_v, sem):
    wid = lax.axis_index("s") * _NC + lax.axis_index("c")
    base = wid * _E_PER_W
    cp_s = pltpu.async_copy(scores_hbm, scores_v, sem)
    cp_r = pltpu.async_copy(edge_hbm.at[pl.ds(base, _E_PER_W)], row_v, sem)
    cp_c = pltpu.async_copy(edge_hbm.at[pl.ds(E + base, _E_PER_W)], col_v, sem)
    cp_s.wait()
    cp_r.wait()
    cp_c.wait()

    def body(i, _):
        off = i * _L
        r = row_v[pl.ds(off, _L)]
        c = col_v[pl.ds(off, _L)]
        sr = plsc.load_gather(scores_v, [r])
        sc = plsc.load_gather(scores_v, [c])
        out_v[pl.ds(off, _L)] = sr * sc
        return 0

    lax.fori_loop(0, _E_PER_W // _L, body, 0)
    pltpu.sync_copy(out_v, out_hbm.at[pl.ds(base, _E_PER_W)])


def kernel(x, edge_index, W, b):
    scores = _compute_scores(x, W, b)
    edge_flat = edge_index.astype(jnp.int32).reshape(2 * E)
    attn = _build_edge_kernel()(scores, edge_flat)
    return attn.reshape(E, 1)


# speedup vs baseline: 119.8662x; 1.1677x over previous
import functools

import jax
import jax.numpy as jnp
from jax import lax
from jax.experimental import pallas as pl
from jax.experimental.pallas import tpu as pltpu
from jax.experimental.pallas import tpu_sc as plsc

N = 10000
E = 320000
D = 128

_NC, _NS, _L = 2, 16, 16
_NW = _NC * _NS
_E_MAIN = 9984
_E_TAIL = E - 31 * _E_MAIN
_UNROLL = 4


def _scores_body(x_ref, w_ref, b_ref, out_ref):
    z = lax.dot_general(
        w_ref[...], x_ref[...],
        dimension_numbers=(((1,), (1,)), ((), ())),
        preferred_element_type=jnp.float32,
    )
    out_ref[...] = jax.nn.sigmoid(z[0] + b_ref[0])


def _compute_scores(x, W, b):
    return pl.pallas_call(
        _scores_body,
        out_shape=jax.ShapeDtypeStruct((N,), jnp.float32),
        in_specs=[
            pl.BlockSpec(memory_space=pltpu.VMEM),
            pl.BlockSpec(memory_space=pltpu.VMEM),
            pl.BlockSpec(memory_space=pltpu.SMEM),
        ],
        out_specs=pl.BlockSpec(memory_space=pltpu.VMEM),
    )(x, W, b)


@functools.cache
def _build_edge_kernel():
    mesh = plsc.VectorSubcoreMesh(core_axis_name="c", subcore_axis_name="s")
    return pl.kernel(
        _edge_body,
        out_type=jax.ShapeDtypeStruct((E,), jnp.float32),
        mesh=mesh,
        scratch_types=[
            pltpu.VMEM((N,), jnp.float32),
            pltpu.VMEM((2, _E_TAIL), jnp.int32),
            pltpu.VMEM((_E_TAIL,), jnp.float32),
            pltpu.SemaphoreType.DMA,
            pltpu.SemaphoreType.DMA,
        ],
        compiler_params=pltpu.CompilerParams(needs_layout_passes=False),
    )


def _edge_body(scores_hbm, edge_hbm, out_hbm, scores_v, rc_v, out_v, sem, sem2):
    wid = lax.axis_index("s") * _NC + lax.axis_index("c")
    is_tail = wid == _NW - 1
    base = wid * _E_MAIN
    cp_s = pltpu.async_copy(scores_hbm, scores_v, sem)
    cp_rc = pltpu.async_copy(
        edge_hbm.at[:, pl.ds(base, _E_MAIN)], rc_v.at[:, pl.ds(0, _E_MAIN)], sem
    )

    @pl.when(is_tail)
    def _():
        pltpu.async_copy(
            edge_hbm.at[:, pl.ds(31 * _E_MAIN + _E_MAIN, _E_TAIL - _E_MAIN)],
            rc_v.at[:, pl.ds(_E_MAIN, _E_TAIL - _E_MAIN)],
            sem2,
        ).wait()

    cp_s.wait()
    cp_rc.wait()

    n_groups = jnp.where(is_tail, _E_TAIL // (_L * _UNROLL), _E_MAIN // (_L * _UNROLL))

    def body(i, _):
        for u in range(_UNROLL):
            off = (i * _UNROLL + u) * _L
            r = rc_v[0, pl.ds(off, _L)]
            c = rc_v[1, pl.ds(off, _L)]
            sr = plsc.load_gather(scores_v, [r])
            sc = plsc.load_gather(scores_v, [c])
            out_v[pl.ds(off, _L)] = sr * sc
        return 0

    lax.fori_loop(0, n_groups, body, 0, unroll=False)
    pltpu.sync_copy(out_v.at[pl.ds(0, _E_MAIN)], out_hbm.at[pl.ds(base, _E_MAIN)])

    @pl.when(is_tail)
    def _():
        pltpu.sync_copy(
            out_v.at[pl.ds(_E_MAIN, _E_TAIL - _E_MAIN)],
            out_hbm.at[pl.ds(31 * _E_MAIN + _E_MAIN, _E_TAIL - _E_MAIN)],
        )


def kernel(x, edge_index, W, b):
    scores = _compute_scores(x, W, b)
    attn = _build_edge_kernel()(scores, edge_index.astype(jnp.int32))
    return attn.reshape(E, 1)


# speedup vs baseline: 139.2367x; 1.1616x over previous
import functools

import jax
import jax.numpy as jnp
from jax import lax
from jax.experimental import pallas as pl
from jax.experimental.pallas import tpu as pltpu
from jax.experimental.pallas import tpu_sc as plsc

N = 10000
E = 320000
D = 128

_NC, _NS, _L = 2, 16, 16
_NW = _NC * _NS
_E_MAIN = 9984
_E_HALF = _E_MAIN // 2
_E_TAIL = E - 31 * _E_MAIN
_E_EXTRA = _E_TAIL - _E_MAIN


def _scores_body(x_ref, w_ref, b_ref, out_ref):
    z = lax.dot_general(
        w_ref[...], x_ref[...],
        dimension_numbers=(((1,), (1,)), ((), ())),
        preferred_element_type=jnp.float32,
    )
    out_ref[...] = jax.nn.sigmoid(z[0] + b_ref[0])


def _compute_scores(x, W, b):
    return pl.pallas_call(
        _scores_body,
        out_shape=jax.ShapeDtypeStruct((N,), jnp.float32),
        in_specs=[
            pl.BlockSpec(memory_space=pltpu.VMEM),
            pl.BlockSpec(memory_space=pltpu.VMEM),
            pl.BlockSpec(memory_space=pltpu.SMEM),
        ],
        out_specs=pl.BlockSpec(memory_space=pltpu.VMEM),
    )(x, W, b)


@functools.cache
def _build_edge_kernel():
    mesh = plsc.VectorSubcoreMesh(core_axis_name="c", subcore_axis_name="s")
    return pl.kernel(
        _edge_body,
        out_type=jax.ShapeDtypeStruct((E,), jnp.float32),
        mesh=mesh,
        scratch_types=[
            pltpu.VMEM((N,), jnp.float32),
            pltpu.VMEM((2, _E_TAIL), jnp.int32),
            pltpu.VMEM((_E_TAIL,), jnp.float32),
            pltpu.SemaphoreType.DMA,
            pltpu.SemaphoreType.DMA,
            pltpu.SemaphoreType.DMA,
        ],
        compiler_params=pltpu.CompilerParams(needs_layout_passes=False),
    )


def _edge_body(scores_hbm, edge_hbm, out_hbm, scores_v, rc_v, out_v, sem, sem_t, sem_st):
    wid = lax.axis_index("s") * _NC + lax.axis_index("c")
    is_tail = wid == _NW - 1
    base = wid * _E_MAIN
    cp_s = pltpu.async_copy(scores_hbm, scores_v, sem)
    cp_i1 = pltpu.async_copy(
        edge_hbm.at[:, pl.ds(base, _E_HALF)], rc_v.at[:, pl.ds(0, _E_HALF)], sem
    )
    cp_i2 = pltpu.async_copy(
        edge_hbm.at[:, pl.ds(base + _E_HALF, _E_HALF)],
        rc_v.at[:, pl.ds(_E_HALF, _E_HALF)],
        sem,
    )

    @pl.when(is_tail)
    def _():
        pltpu.async_copy(
            edge_hbm.at[:, pl.ds(31 * _E_MAIN + _E_MAIN, _E_EXTRA)],
            rc_v.at[:, pl.ds(_E_MAIN, _E_EXTRA)],
            sem_t,
        )

    def group(off):
        r = rc_v[0, pl.ds(off, _L)]
        c = rc_v[1, pl.ds(off, _L)]
        sr = plsc.load_gather(scores_v, [r])
        sc = plsc.load_gather(scores_v, [c])
        out_v[pl.ds(off, _L)] = sr * sc

    cp_s.wait()
    cp_i1.wait()

    plsc.parallel_loop(0, _E_HALF, _L, unroll=8)(group)
    cp_o1 = pltpu.async_copy(
        out_v.at[pl.ds(0, _E_HALF)], out_hbm.at[pl.ds(base, _E_HALF)], sem_st
    )

    cp_i2.wait()
    plsc.parallel_loop(_E_HALF, _E_MAIN, _L, unroll=8)(group)
    cp_o2 = pltpu.async_copy(
        out_v.at[pl.ds(_E_HALF, _E_HALF)],
        out_hbm.at[pl.ds(base + _E_HALF, _E_HALF)],
        sem_st,
    )

    @pl.when(is_tail)
    def _():
        pltpu.make_async_copy(
            edge_hbm.at[:, pl.ds(31 * _E_MAIN + _E_MAIN, _E_EXTRA)],
            rc_v.at[:, pl.ds(_E_MAIN, _E_EXTRA)],
            sem_t,
        ).wait()
        plsc.parallel_loop(_E_MAIN, _E_TAIL, _L, unroll=8)(group)
        pltpu.sync_copy(
            out_v.at[pl.ds(_E_MAIN, _E_EXTRA)],
            out_hbm.at[pl.ds(31 * _E_MAIN + _E_MAIN, _E_EXTRA)],
        )

    cp_o1.wait()
    cp_o2.wait()


def kernel(x, edge_index, W, b):
    scores = _compute_scores(x, W, b)
    attn = _build_edge_kernel()(scores, edge_index.astype(jnp.int32))
    return attn.reshape(E, 1)


# speedup vs baseline: 170.6225x; 1.2254x over previous
import functools

import jax
import jax.numpy as jnp
from jax import lax
from jax.experimental import pallas as pl
from jax.experimental.pallas import tpu as pltpu
from jax.experimental.pallas import tpu_sc as plsc

N = 10000
E = 320000
D = 128

_NC, _NS, _L = 2, 16, 16
_NW = _NC * _NS
_R_MAIN = 78
_R_TAIL = E // D - 31 * _R_MAIN
_R_HALF = _R_MAIN // 2
_E_MAIN = _R_MAIN * D
_E_HALF = _R_HALF * D
_E_TAIL = _R_TAIL * D
_GROUPS_PER_ROW = D // _L


def _scores_body(x_ref, w_ref, b_ref, out_ref):
    z = lax.dot_general(
        w_ref[...], x_ref[...],
        dimension_numbers=(((1,), (1,)), ((), ())),
        preferred_element_type=jnp.float32,
    )
    out_ref[...] = jax.nn.sigmoid(z[0] + b_ref[0])


def _compute_scores(x, W, b):
    return pl.pallas_call(
        _scores_body,
        out_shape=jax.ShapeDtypeStruct((N,), jnp.float32),
        in_specs=[
            pl.BlockSpec(memory_space=pltpu.VMEM),
            pl.BlockSpec(memory_space=pltpu.VMEM),
            pl.BlockSpec(memory_space=pltpu.SMEM),
        ],
        out_specs=pl.BlockSpec(memory_space=pltpu.VMEM),
    )(x, W, b)


@functools.cache
def _build_edge_kernel():
    mesh = plsc.VectorSubcoreMesh(core_axis_name="c", subcore_axis_name="s")
    return pl.kernel(
        _edge_body,
        out_type=jax.ShapeDtypeStruct((E // D, 1, D), jnp.float32),
        mesh=mesh,
        scratch_types=[
            pltpu.VMEM((N,), jnp.float32),
            pltpu.VMEM((2, _E_TAIL), jnp.int32),
            pltpu.VMEM((_R_TAIL, 1, D), jnp.float32),
            pltpu.SemaphoreType.DMA,
            pltpu.SemaphoreType.DMA,
            pltpu.SemaphoreType.DMA,
        ],
        compiler_params=pltpu.CompilerParams(needs_layout_passes=False),
    )


def _edge_body(scores_hbm, edge_hbm, out_hbm, scores_v, rc_v, out_v, sem, sem_t, sem_st):
    wid = lax.axis_index("s") * _NC + lax.axis_index("c")
    is_tail = wid == _NW - 1
    base = wid * _E_MAIN
    row_base = wid * _R_MAIN
    cp_s = pltpu.async_copy(scores_hbm, scores_v, sem)
    cp_i1 = pltpu.async_copy(
        edge_hbm.at[:, pl.ds(base, _E_HALF)], rc_v.at[:, pl.ds(0, _E_HALF)], sem
    )
    cp_i2 = pltpu.async_copy(
        edge_hbm.at[:, pl.ds(base + _E_HALF, _E_HALF)],
        rc_v.at[:, pl.ds(_E_HALF, _E_HALF)],
        sem,
    )

    @pl.when(is_tail)
    def _():
        pltpu.async_copy(
            edge_hbm.at[:, pl.ds(32 * _E_MAIN, _E_TAIL - _E_MAIN)],
            rc_v.at[:, pl.ds(_E_MAIN, _E_TAIL - _E_MAIN)],
            sem_t,
        )

    def row_body(r):
        for j in range(_GROUPS_PER_ROW):
            off = r * D + j * _L
            rr = rc_v[0, pl.ds(off, _L)]
            cc = rc_v[1, pl.ds(off, _L)]
            sr = plsc.load_gather(scores_v, [rr])
            sc = plsc.load_gather(scores_v, [cc])
            out_v[r, 0, pl.ds(j * _L, _L)] = sr * sc

    cp_s.wait()
    cp_i1.wait()

    plsc.parallel_loop(0, _R_HALF, 1)(row_body)
    cp_o1 = pltpu.async_copy(
        out_v.at[pl.ds(0, _R_HALF), :, :], out_hbm.at[pl.ds(row_base, _R_HALF), :, :], sem_st
    )

    cp_i2.wait()
    plsc.parallel_loop(_R_HALF, _R_MAIN, 1)(row_body)
    cp_o2 = pltpu.async_copy(
        out_v.at[pl.ds(_R_HALF, _R_HALF), :, :],
        out_hbm.at[pl.ds(row_base + _R_HALF, _R_HALF), :, :],
        sem_st,
    )

    @pl.when(is_tail)
    def _():
        pltpu.make_async_copy(
            edge_hbm.at[:, pl.ds(32 * _E_MAIN, _E_TAIL - _E_MAIN)],
            rc_v.at[:, pl.ds(_E_MAIN, _E_TAIL - _E_MAIN)],
            sem_t,
        ).wait()
        plsc.parallel_loop(_R_MAIN, _R_TAIL, 1)(row_body)
        pltpu.sync_copy(
            out_v.at[pl.ds(_R_MAIN, _R_TAIL - _R_MAIN), :, :],
            out_hbm.at[pl.ds(31 * _R_MAIN + _R_MAIN, _R_TAIL - _R_MAIN), :, :],
        )

    cp_o1.wait()
    cp_o2.wait()


def kernel(x, edge_index, W, b):
    scores = _compute_scores(x, W, b)
    attn = _build_edge_kernel()(scores, edge_index.astype(jnp.int32))
    return attn.reshape(E, 1)
